# baseline (speedup 1.0000x reference)
import functools

import jax
import jax.numpy as jnp
from jax import lax
from jax.experimental import pallas as pl
from jax.experimental.pallas import tpu as pltpu
from jax.experimental.pallas import tpu_sc as plsc

B, C, D = 1024, 512, 64

BB = 32
NSC = 256
BT = B - NSC
GT = BT // BB


def _tc_body(xt_ref, masked_ref, idx_ref):
    xt = xt_ref[...]
    s = jnp.sum(xt * xt, axis=1)
    smax = jnp.max(s, axis=1, keepdims=True)
    c_iota = lax.broadcasted_iota(jnp.int32, (BB, C), 1)
    idx = jnp.min(jnp.where(s >= smax, c_iota, C), axis=1, keepdims=True)
    oh = (c_iota == idx).astype(jnp.float32)
    r = lax.dot_general(oh, xt.reshape(BB * D, C),
                        dimension_numbers=(((1,), (1,)), ((), ())),
                        preferred_element_type=jnp.float32)
    b_col = lax.broadcasted_iota(jnp.int32, (BB, 1), 0)
    acc = jnp.zeros((BB, D), jnp.float32)
    for j in range(BB):
        acc = acc + jnp.where(b_col == j, r[:, j * D:(j + 1) * D], 0.0)
    masked_ref[...] = acc
    idx_ref[...] = idx


def _tc_kernel(xt):
    return pl.pallas_call(
        _tc_body,
        grid=(GT,),
        in_specs=[pl.BlockSpec((BB, D, C), lambda i: (i, 0, 0))],
        out_specs=[
            pl.BlockSpec((BB, D), lambda i: (i, 0)),
            pl.BlockSpec((BB, 1), lambda i: (i, 0)),
        ],
        out_shape=[
            jax.ShapeDtypeStruct((BT, D), jnp.float32),
            jax.ShapeDtypeStruct((BT, 1), jnp.int32),
        ],
    )(xt)


_info = plsc.get_sparse_core_info()
NC, NS, L = _info.num_cores, _info.num_subcores, _info.num_lanes
NW = NC * NS
RW = NSC // NW

_mesh = plsc.VectorSubcoreMesh(core_axis_name="c", subcore_axis_name="s")

_GDN = lax.GatherDimensionNumbers(
    offset_dims=(), collapsed_slice_dims=(0,), start_index_map=(0,))


def _lane_perm(v, perm):
    return lax.gather(v, perm[:, None], _GDN, (1,),
                      mode=lax.GatherScatterMode.PROMISE_IN_BOUNDS)


def _row_argmax(slab_v, iota16):
    def chunk_body(g, bc):
        best_v, best_c = bc
        c0 = pl.multiple_of(g * L, L)
        a0 = jnp.zeros((L,), jnp.float32)
        a1 = jnp.zeros((L,), jnp.float32)
        a2 = jnp.zeros((L,), jnp.float32)
        a3 = jnp.zeros((L,), jnp.float32)
        for d in range(0, D, 4):
            v0 = slab_v[0, d, pl.ds(c0, L)]
            v1 = slab_v[0, d + 1, pl.ds(c0, L)]
            v2 = slab_v[0, d + 2, pl.ds(c0, L)]
            v3 = slab_v[0, d + 3, pl.ds(c0, L)]
            a0 = a0 + v0 * v0
            a1 = a1 + v1 * v1
            a2 = a2 + v2 * v2
            a3 = a3 + v3 * v3
        acc = (a0 + a1) + (a2 + a3)
        upd = acc > best_v
        best_v = jnp.where(upd, acc, best_v)
        best_c = jnp.where(upd, c0 + iota16, best_c)
        return best_v, best_c

    best_v, best_c = lax.fori_loop(
        0, C // L, chunk_body,
        (jnp.full((L,), -1.0, jnp.float32), jnp.zeros((L,), jnp.int32)))

    m = best_v
    for sh in (8, 4, 2, 1):
        perm = (iota16 + sh) % L
        m = jnp.maximum(m, _lane_perm(m, perm))
    cand = jnp.where(best_v >= m, best_c, C)
    for sh in (8, 4, 2, 1):
        perm = (iota16 + sh) % L
        cand = jnp.minimum(cand, _lane_perm(cand, perm))
    return cand


def _extract_column(slab_v, cand, row_v, iota16):
    c_s = cand[0]
    c_al = pl.multiple_of((c_s // L) * L, L)
    perm = iota16 * 0 + (c_s % L)
    for k in range(D // L):
        out_k = jnp.zeros((L,), jnp.float32)
        for dd in range(L):
            v16 = slab_v[0, k * L + dd, pl.ds(c_al, L)]
            w = _lane_perm(v16, perm)
            out_k = jnp.where(iota16 == dd, w, out_k)
        row_v[pl.ds(k * L, L)] = out_k


@functools.partial(
    pl.kernel,
    mesh=_mesh,
    out_type=[
        jax.ShapeDtypeStruct((NSC * D,), jnp.float32),
        jax.ShapeDtypeStruct((NW * L,), jnp.int32),
    ],
    scratch_types=[
        pltpu.VMEM((1, D, C), jnp.float32),
        pltpu.VMEM((1, D, C), jnp.float32),
        pltpu.VMEM((2 * D,), jnp.float32),
        pltpu.VMEM((L,), jnp.int32),
        pltpu.SemaphoreType.DMA,
        pltpu.SemaphoreType.DMA,
    ],
    compiler_params=pltpu.CompilerParams(use_tc_tiling_on_sc=True),
)
def _sc_kernel(x_hbm, masked_hbm, idx_hbm, slab0, slab1, pair_v,
               idx_stage, sem0, sem1):
    wid = lax.axis_index("s") * NC + lax.axis_index("c")
    base = BT + wid * RW
    last = base + RW - 1
    iota16 = lax.broadcasted_iota(jnp.int32, (L,), 0)

    def _wait(slab, sem):
        pltpu.make_async_copy(x_hbm.at[pl.ds(0, 1)], slab, sem).wait()

    pltpu.async_copy(x_hbm.at[pl.ds(base, 1)], slab0, sem0)
    pltpu.async_copy(x_hbm.at[pl.ds(base + 1, 1)], slab1, sem1)

    def pair_body(t, idx_vec):
        b0 = base + 2 * t
        _wait(slab0, sem0)
        cand0 = _row_argmax(slab0, iota16)
        _extract_column(slab0, cand0, pair_v.at[pl.ds(0, D)], iota16)
        nxt = jnp.minimum(b0 + 2, last)
        pltpu.async_copy(x_hbm.at[pl.ds(nxt, 1)], slab0, sem0)
        _wait(slab1, sem1)
        cand1 = _row_argmax(slab1, iota16)
        _extract_column(slab1, cand1, pair_v.at[pl.ds(D, D)], iota16)
        nxt = jnp.minimum(b0 + 3, last)
        pltpu.async_copy(x_hbm.at[pl.ds(nxt, 1)], slab1, sem1)

        pltpu.sync_copy(pair_v, masked_hbm.at[pl.ds((b0 - BT) * D, 2 * D)])

        idx_vec = jnp.where(iota16 == 2 * t, cand0, idx_vec)
        return jnp.where(iota16 == 2 * t + 1, cand1, idx_vec)

    idx_vec = lax.fori_loop(0, RW // 2, pair_body, jnp.zeros((L,), jnp.int32))
    _wait(slab0, sem0)
    _wait(slab1, sem1)

    idx_stage[pl.ds(0, L)] = idx_vec
    pltpu.sync_copy(idx_stage, idx_hbm.at[pl.ds(wid * L, L)])


@jax.jit
def kernel(x):
    xt = jnp.transpose(x, (0, 2, 1))
    sc_masked_flat, sc_idx_pad = _sc_kernel(xt)
    tc_masked, tc_idx = _tc_kernel(xt)
    masked = jnp.concatenate(
        [tc_masked, sc_masked_flat.reshape(NSC, D)], axis=0)
    sc_idx = sc_idx_pad.reshape(NW, L)[:, :RW].reshape(NSC)
    idx = jnp.concatenate([tc_idx.reshape(BT), sc_idx], axis=0)
    return masked[:, None, :], idx


# speedup vs baseline: 1.0601x; 1.0601x over previous
import functools

import jax
import jax.numpy as jnp
from jax import lax
from jax.experimental import pallas as pl
from jax.experimental.pallas import tpu as pltpu
from jax.experimental.pallas import tpu_sc as plsc

B, C, D = 1024, 512, 64

BB = 32
NSC = 384
BT = B - NSC
GT = BT // BB


def _tc_body(xt_ref, masked_ref, idx_ref):
    xt = xt_ref[...]
    s = jnp.sum(xt * xt, axis=1)
    smax = jnp.max(s, axis=1, keepdims=True)
    c_iota = lax.broadcasted_iota(jnp.int32, (BB, C), 1)
    idx = jnp.min(jnp.where(s >= smax, c_iota, C), axis=1, keepdims=True)
    oh = (c_iota == idx).astype(jnp.float32)
    r = lax.dot_general(oh, xt.reshape(BB * D, C),
                        dimension_numbers=(((1,), (1,)), ((), ())),
                        preferred_element_type=jnp.float32)
    b_col = lax.broadcasted_iota(jnp.int32, (BB, 1), 0)
    acc = jnp.zeros((BB, D), jnp.float32)
    for j in range(BB):
        acc = acc + jnp.where(b_col == j, r[:, j * D:(j + 1) * D], 0.0)
    masked_ref[...] = acc
    idx_ref[...] = idx


def _tc_kernel(xt):
    return pl.pallas_call(
        _tc_body,
        grid=(GT,),
        in_specs=[pl.BlockSpec((BB, D, C), lambda i: (i, 0, 0))],
        out_specs=[
            pl.BlockSpec((BB, D), lambda i: (i, 0)),
            pl.BlockSpec((BB, 1), lambda i: (i, 0)),
        ],
        out_shape=[
            jax.ShapeDtypeStruct((BT, D), jnp.float32),
            jax.ShapeDtypeStruct((BT, 1), jnp.int32),
        ],
    )(xt)


_info = plsc.get_sparse_core_info()
NC, NS, L = _info.num_cores, _info.num_subcores, _info.num_lanes
NW = NC * NS
RW = NSC // NW

_mesh = plsc.VectorSubcoreMesh(core_axis_name="c", subcore_axis_name="s")

_GDN = lax.GatherDimensionNumbers(
    offset_dims=(), collapsed_slice_dims=(0,), start_index_map=(0,))


def _lane_perm(v, perm):
    return lax.gather(v, perm[:, None], _GDN, (1,),
                      mode=lax.GatherScatterMode.PROMISE_IN_BOUNDS)


def _row_argmax(slab_v, iota16):
    def chunk_body(g, bc):
        best_v, best_c = bc
        c0 = pl.multiple_of(g * L, L)
        a0 = jnp.zeros((L,), jnp.float32)
        a1 = jnp.zeros((L,), jnp.float32)
        a2 = jnp.zeros((L,), jnp.float32)
        a3 = jnp.zeros((L,), jnp.float32)
        for d in range(0, D, 4):
            v0 = slab_v[0, d, pl.ds(c0, L)]
            v1 = slab_v[0, d + 1, pl.ds(c0, L)]
            v2 = slab_v[0, d + 2, pl.ds(c0, L)]
            v3 = slab_v[0, d + 3, pl.ds(c0, L)]
            a0 = a0 + v0 * v0
            a1 = a1 + v1 * v1
            a2 = a2 + v2 * v2
            a3 = a3 + v3 * v3
        acc = (a0 + a1) + (a2 + a3)
        upd = acc > best_v
        best_v = jnp.where(upd, acc, best_v)
        best_c = jnp.where(upd, c0 + iota16, best_c)
        return best_v, best_c

    best_v, best_c = lax.fori_loop(
        0, C // L, chunk_body,
        (jnp.full((L,), -1.0, jnp.float32), jnp.zeros((L,), jnp.int32)))

    m = best_v
    for sh in (8, 4, 2, 1):
        perm = (iota16 + sh) % L
        m = jnp.maximum(m, _lane_perm(m, perm))
    cand = jnp.where(best_v >= m, best_c, C)
    for sh in (8, 4, 2, 1):
        perm = (iota16 + sh) % L
        cand = jnp.minimum(cand, _lane_perm(cand, perm))
    return cand


def _extract_column(slab_v, cand, row_v, iota16):
    c_s = cand[0]
    c_al = pl.multiple_of((c_s // L) * L, L)
    perm = iota16 * 0 + (c_s % L)
    for k in range(D // L):
        out_k = jnp.zeros((L,), jnp.float32)
        for dd in range(L):
            v16 = slab_v[0, k * L + dd, pl.ds(c_al, L)]
            w = _lane_perm(v16, perm)
            out_k = jnp.where(iota16 == dd, w, out_k)
        row_v[pl.ds(k * L, L)] = out_k


@functools.partial(
    pl.kernel,
    mesh=_mesh,
    out_type=[
        jax.ShapeDtypeStruct((NSC * D,), jnp.float32),
        jax.ShapeDtypeStruct((NW * L,), jnp.int32),
    ],
    scratch_types=[
        pltpu.VMEM((1, D, C), jnp.float32),
        pltpu.VMEM((1, D, C), jnp.float32),
        pltpu.VMEM((2 * D,), jnp.float32),
        pltpu.VMEM((L,), jnp.int32),
        pltpu.SemaphoreType.DMA,
        pltpu.SemaphoreType.DMA,
    ],
    compiler_params=pltpu.CompilerParams(use_tc_tiling_on_sc=True),
)
def _sc_kernel(x_hbm, masked_hbm, idx_hbm, slab0, slab1, pair_v,
               idx_stage, sem0, sem1):
    wid = lax.axis_index("s") * NC + lax.axis_index("c")
    base = BT + wid * RW
    last = base + RW - 1
    iota16 = lax.broadcasted_iota(jnp.int32, (L,), 0)

    def _wait(slab, sem):
        pltpu.make_async_copy(x_hbm.at[pl.ds(0, 1)], slab, sem).wait()

    pltpu.async_copy(x_hbm.at[pl.ds(base, 1)], slab0, sem0)
    pltpu.async_copy(x_hbm.at[pl.ds(base + 1, 1)], slab1, sem1)

    def pair_body(t, idx_vec):
        b0 = base + 2 * t
        _wait(slab0, sem0)
        cand0 = _row_argmax(slab0, iota16)
        _extract_column(slab0, cand0, pair_v.at[pl.ds(0, D)], iota16)
        nxt = jnp.minimum(b0 + 2, last)
        pltpu.async_copy(x_hbm.at[pl.ds(nxt, 1)], slab0, sem0)
        _wait(slab1, sem1)
        cand1 = _row_argmax(slab1, iota16)
        _extract_column(slab1, cand1, pair_v.at[pl.ds(D, D)], iota16)
        nxt = jnp.minimum(b0 + 3, last)
        pltpu.async_copy(x_hbm.at[pl.ds(nxt, 1)], slab1, sem1)

        pltpu.sync_copy(pair_v, masked_hbm.at[pl.ds((b0 - BT) * D, 2 * D)])

        idx_vec = jnp.where(iota16 == 2 * t, cand0, idx_vec)
        return jnp.where(iota16 == 2 * t + 1, cand1, idx_vec)

    idx_vec = lax.fori_loop(0, RW // 2, pair_body, jnp.zeros((L,), jnp.int32))
    _wait(slab0, sem0)
    _wait(slab1, sem1)

    idx_stage[pl.ds(0, L)] = idx_vec
    pltpu.sync_copy(idx_stage, idx_hbm.at[pl.ds(wid * L, L)])


@jax.jit
def kernel(x):
    xt = jnp.transpose(x, (0, 2, 1))
    sc_masked_flat, sc_idx_pad = _sc_kernel(xt)
    tc_masked, tc_idx = _tc_kernel(xt)
    masked = jnp.concatenate(
        [tc_masked, sc_masked_flat.reshape(NSC, D)], axis=0)
    sc_idx = sc_idx_pad.reshape(NW, L)[:, :RW].reshape(NSC)
    idx = jnp.concatenate([tc_idx.reshape(BT), sc_idx], axis=0)
    return masked[:, None, :], idx


# speedup vs baseline: 1.0871x; 1.0254x over previous
import functools

import jax
import jax.numpy as jnp
from jax import lax
from jax.experimental import pallas as pl
from jax.experimental.pallas import tpu as pltpu
from jax.experimental.pallas import tpu_sc as plsc

B, C, D = 1024, 512, 64

BB = 32
NSC = 448
BT = B - NSC
GT = BT // BB


def _tc_body(xt_ref, masked_ref, idx_ref):
    xt = xt_ref[...]
    s = jnp.sum(xt * xt, axis=1)
    smax = jnp.max(s, axis=1, keepdims=True)
    c_iota = lax.broadcasted_iota(jnp.int32, (BB, C), 1)
    idx = jnp.min(jnp.where(s >= smax, c_iota, C), axis=1, keepdims=True)
    oh = (c_iota == idx).astype(jnp.float32)
    r = lax.dot_general(oh, xt.reshape(BB * D, C),
                        dimension_numbers=(((1,), (1,)), ((), ())),
                        preferred_element_type=jnp.float32)
    b_col = lax.broadcasted_iota(jnp.int32, (BB, 1), 0)
    acc = jnp.zeros((BB, D), jnp.float32)
    for j in range(BB):
        acc = acc + jnp.where(b_col == j, r[:, j * D:(j + 1) * D], 0.0)
    masked_ref[...] = acc
    idx_ref[...] = idx


def _tc_kernel(xt):
    return pl.pallas_call(
        _tc_body,
        grid=(GT,),
        in_specs=[pl.BlockSpec((BB, D, C), lambda i: (i, 0, 0))],
        out_specs=[
            pl.BlockSpec((BB, D), lambda i: (i, 0)),
            pl.BlockSpec((BB, 1), lambda i: (i, 0)),
        ],
        out_shape=[
            jax.ShapeDtypeStruct((BT, D), jnp.float32),
            jax.ShapeDtypeStruct((BT, 1), jnp.int32),
        ],
    )(xt)


_info = plsc.get_sparse_core_info()
NC, NS, L = _info.num_cores, _info.num_subcores, _info.num_lanes
NW = NC * NS
RW = NSC // NW

_mesh = plsc.VectorSubcoreMesh(core_axis_name="c", subcore_axis_name="s")

_GDN = lax.GatherDimensionNumbers(
    offset_dims=(), collapsed_slice_dims=(0,), start_index_map=(0,))


def _lane_perm(v, perm):
    return lax.gather(v, perm[:, None], _GDN, (1,),
                      mode=lax.GatherScatterMode.PROMISE_IN_BOUNDS)


def _row_argmax(slab_v, iota16):
    def chunk_body(g, bc):
        best_v, best_c = bc
        c0 = pl.multiple_of(g * L, L)
        a0 = jnp.zeros((L,), jnp.float32)
        a1 = jnp.zeros((L,), jnp.float32)
        a2 = jnp.zeros((L,), jnp.float32)
        a3 = jnp.zeros((L,), jnp.float32)
        for d in range(0, D, 4):
            v0 = slab_v[0, d, pl.ds(c0, L)]
            v1 = slab_v[0, d + 1, pl.ds(c0, L)]
            v2 = slab_v[0, d + 2, pl.ds(c0, L)]
            v3 = slab_v[0, d + 3, pl.ds(c0, L)]
            a0 = a0 + v0 * v0
            a1 = a1 + v1 * v1
            a2 = a2 + v2 * v2
            a3 = a3 + v3 * v3
        acc = (a0 + a1) + (a2 + a3)
        upd = acc > best_v
        best_v = jnp.where(upd, acc, best_v)
        best_c = jnp.where(upd, c0 + iota16, best_c)
        return best_v, best_c

    best_v, best_c = lax.fori_loop(
        0, C // L, chunk_body,
        (jnp.full((L,), -1.0, jnp.float32), jnp.zeros((L,), jnp.int32)))

    m = best_v
    for sh in (8, 4, 2, 1):
        perm = (iota16 + sh) % L
        m = jnp.maximum(m, _lane_perm(m, perm))
    cand = jnp.where(best_v >= m, best_c, C)
    for sh in (8, 4, 2, 1):
        perm = (iota16 + sh) % L
        cand = jnp.minimum(cand, _lane_perm(cand, perm))
    return cand


def _extract_column(slab_v, cand, row_v, iota16):
    c_s = cand[0]
    c_al = pl.multiple_of((c_s // L) * L, L)
    perm = iota16 * 0 + (c_s % L)
    for k in range(D // L):
        out_k = jnp.zeros((L,), jnp.float32)
        for dd in range(L):
            v16 = slab_v[0, k * L + dd, pl.ds(c_al, L)]
            w = _lane_perm(v16, perm)
            out_k = jnp.where(iota16 == dd, w, out_k)
        row_v[pl.ds(k * L, L)] = out_k


@functools.partial(
    pl.kernel,
    mesh=_mesh,
    out_type=[
        jax.ShapeDtypeStruct((NSC * D,), jnp.float32),
        jax.ShapeDtypeStruct((NW * L,), jnp.int32),
    ],
    scratch_types=[
        pltpu.VMEM((1, D, C), jnp.float32),
        pltpu.VMEM((1, D, C), jnp.float32),
        pltpu.VMEM((2 * D,), jnp.float32),
        pltpu.VMEM((L,), jnp.int32),
        pltpu.SemaphoreType.DMA,
        pltpu.SemaphoreType.DMA,
    ],
    compiler_params=pltpu.CompilerParams(use_tc_tiling_on_sc=True),
)
def _sc_kernel(x_hbm, masked_hbm, idx_hbm, slab0, slab1, pair_v,
               idx_stage, sem0, sem1):
    wid = lax.axis_index("s") * NC + lax.axis_index("c")
    base = BT + wid * RW
    last = base + RW - 1
    iota16 = lax.broadcasted_iota(jnp.int32, (L,), 0)

    def _wait(slab, sem):
        pltpu.make_async_copy(x_hbm.at[pl.ds(0, 1)], slab, sem).wait()

    pltpu.async_copy(x_hbm.at[pl.ds(base, 1)], slab0, sem0)
    pltpu.async_copy(x_hbm.at[pl.ds(base + 1, 1)], slab1, sem1)

    def pair_body(t, idx_vec):
        b0 = base + 2 * t
        _wait(slab0, sem0)
        cand0 = _row_argmax(slab0, iota16)
        _extract_column(slab0, cand0, pair_v.at[pl.ds(0, D)], iota16)
        nxt = jnp.minimum(b0 + 2, last)
        pltpu.async_copy(x_hbm.at[pl.ds(nxt, 1)], slab0, sem0)
        _wait(slab1, sem1)
        cand1 = _row_argmax(slab1, iota16)
        _extract_column(slab1, cand1, pair_v.at[pl.ds(D, D)], iota16)
        nxt = jnp.minimum(b0 + 3, last)
        pltpu.async_copy(x_hbm.at[pl.ds(nxt, 1)], slab1, sem1)

        pltpu.sync_copy(pair_v, masked_hbm.at[pl.ds((b0 - BT) * D, 2 * D)])

        idx_vec = jnp.where(iota16 == 2 * t, cand0, idx_vec)
        return jnp.where(iota16 == 2 * t + 1, cand1, idx_vec)

    idx_vec = lax.fori_loop(0, RW // 2, pair_body, jnp.zeros((L,), jnp.int32))
    _wait(slab0, sem0)
    _wait(slab1, sem1)

    idx_stage[pl.ds(0, L)] = idx_vec
    pltpu.sync_copy(idx_stage, idx_hbm.at[pl.ds(wid * L, L)])


@jax.jit
def kernel(x):
    xt = jnp.transpose(x, (0, 2, 1))
    sc_masked_flat, sc_idx_pad = _sc_kernel(xt)
    tc_masked, tc_idx = _tc_kernel(xt)
    masked = jnp.concatenate(
        [tc_masked, sc_masked_flat.reshape(NSC, D)], axis=0)
    sc_idx = sc_idx_pad.reshape(NW, L)[:, :RW].reshape(NSC)
    idx = jnp.concatenate([tc_idx.reshape(BT), sc_idx], axis=0)
    return masked[:, None, :], idx


# speedup vs baseline: 1.1140x; 1.0248x over previous
import functools

import jax
import jax.numpy as jnp
from jax import lax
from jax.experimental import pallas as pl
from jax.experimental.pallas import tpu as pltpu
from jax.experimental.pallas import tpu_sc as plsc

B, C, D = 1024, 512, 64

BB = 32
NSC = 512
BT = B - NSC
GT = BT // BB


def _tc_body(xt_ref, masked_ref, idx_ref):
    xt = xt_ref[...]
    s = jnp.sum(xt * xt, axis=1)
    smax = jnp.max(s, axis=1, keepdims=True)
    c_iota = lax.broadcasted_iota(jnp.int32, (BB, C), 1)
    idx = jnp.min(jnp.where(s >= smax, c_iota, C), axis=1, keepdims=True)
    oh = (c_iota == idx).astype(jnp.float32)
    r = lax.dot_general(oh, xt.reshape(BB * D, C),
                        dimension_numbers=(((1,), (1,)), ((), ())),
                        preferred_element_type=jnp.float32)
    b_col = lax.broadcasted_iota(jnp.int32, (BB, 1), 0)
    acc = jnp.zeros((BB, D), jnp.float32)
    for j in range(BB):
        acc = acc + jnp.where(b_col == j, r[:, j * D:(j + 1) * D], 0.0)
    masked_ref[...] = acc
    idx_ref[...] = idx


def _tc_kernel(xt):
    return pl.pallas_call(
        _tc_body,
        grid=(GT,),
        in_specs=[pl.BlockSpec((BB, D, C), lambda i: (i, 0, 0))],
        out_specs=[
            pl.BlockSpec((BB, D), lambda i: (i, 0)),
            pl.BlockSpec((BB, 1), lambda i: (i, 0)),
        ],
        out_shape=[
            jax.ShapeDtypeStruct((BT, D), jnp.float32),
            jax.ShapeDtypeStruct((BT, 1), jnp.int32),
        ],
    )(xt)


_info = plsc.get_sparse_core_info()
NC, NS, L = _info.num_cores, _info.num_subcores, _info.num_lanes
NW = NC * NS
RW = NSC // NW

_mesh = plsc.VectorSubcoreMesh(core_axis_name="c", subcore_axis_name="s")

_GDN = lax.GatherDimensionNumbers(
    offset_dims=(), collapsed_slice_dims=(0,), start_index_map=(0,))


def _lane_perm(v, perm):
    return lax.gather(v, perm[:, None], _GDN, (1,),
                      mode=lax.GatherScatterMode.PROMISE_IN_BOUNDS)


def _row_argmax(slab_v, iota16):
    def chunk_body(g, bc):
        best_v, best_c = bc
        c0 = pl.multiple_of(g * L, L)
        a0 = jnp.zeros((L,), jnp.float32)
        a1 = jnp.zeros((L,), jnp.float32)
        a2 = jnp.zeros((L,), jnp.float32)
        a3 = jnp.zeros((L,), jnp.float32)
        for d in range(0, D, 4):
            v0 = slab_v[0, d, pl.ds(c0, L)]
            v1 = slab_v[0, d + 1, pl.ds(c0, L)]
            v2 = slab_v[0, d + 2, pl.ds(c0, L)]
            v3 = slab_v[0, d + 3, pl.ds(c0, L)]
            a0 = a0 + v0 * v0
            a1 = a1 + v1 * v1
            a2 = a2 + v2 * v2
            a3 = a3 + v3 * v3
        acc = (a0 + a1) + (a2 + a3)
        upd = acc > best_v
        best_v = jnp.where(upd, acc, best_v)
        best_c = jnp.where(upd, c0 + iota16, best_c)
        return best_v, best_c

    best_v, best_c = lax.fori_loop(
        0, C // L, chunk_body,
        (jnp.full((L,), -1.0, jnp.float32), jnp.zeros((L,), jnp.int32)))

    m = best_v
    for sh in (8, 4, 2, 1):
        perm = (iota16 + sh) % L
        m = jnp.maximum(m, _lane_perm(m, perm))
    cand = jnp.where(best_v >= m, best_c, C)
    for sh in (8, 4, 2, 1):
        perm = (iota16 + sh) % L
        cand = jnp.minimum(cand, _lane_perm(cand, perm))
    return cand


def _extract_column(slab_v, cand, row_v, iota16):
    c_s = cand[0]
    c_al = pl.multiple_of((c_s // L) * L, L)
    perm = iota16 * 0 + (c_s % L)
    for k in range(D // L):
        out_k = jnp.zeros((L,), jnp.float32)
        for dd in range(L):
            v16 = slab_v[0, k * L + dd, pl.ds(c_al, L)]
            w = _lane_perm(v16, perm)
            out_k = jnp.where(iota16 == dd, w, out_k)
        row_v[pl.ds(k * L, L)] = out_k


@functools.partial(
    pl.kernel,
    mesh=_mesh,
    out_type=[
        jax.ShapeDtypeStruct((NSC * D,), jnp.float32),
        jax.ShapeDtypeStruct((NW * L,), jnp.int32),
    ],
    scratch_types=[
        pltpu.VMEM((1, D, C), jnp.float32),
        pltpu.VMEM((1, D, C), jnp.float32),
        pltpu.VMEM((2 * D,), jnp.float32),
        pltpu.VMEM((L,), jnp.int32),
        pltpu.SemaphoreType.DMA,
        pltpu.SemaphoreType.DMA,
    ],
    compiler_params=pltpu.CompilerParams(use_tc_tiling_on_sc=True),
)
def _sc_kernel(x_hbm, masked_hbm, idx_hbm, slab0, slab1, pair_v,
               idx_stage, sem0, sem1):
    wid = lax.axis_index("s") * NC + lax.axis_index("c")
    base = BT + wid * RW
    last = base + RW - 1
    iota16 = lax.broadcasted_iota(jnp.int32, (L,), 0)

    def _wait(slab, sem):
        pltpu.make_async_copy(x_hbm.at[pl.ds(0, 1)], slab, sem).wait()

    pltpu.async_copy(x_hbm.at[pl.ds(base, 1)], slab0, sem0)
    pltpu.async_copy(x_hbm.at[pl.ds(base + 1, 1)], slab1, sem1)

    def pair_body(t, idx_vec):
        b0 = base + 2 * t
        _wait(slab0, sem0)
        cand0 = _row_argmax(slab0, iota16)
        _extract_column(slab0, cand0, pair_v.at[pl.ds(0, D)], iota16)
        nxt = jnp.minimum(b0 + 2, last)
        pltpu.async_copy(x_hbm.at[pl.ds(nxt, 1)], slab0, sem0)
        _wait(slab1, sem1)
        cand1 = _row_argmax(slab1, iota16)
        _extract_column(slab1, cand1, pair_v.at[pl.ds(D, D)], iota16)
        nxt = jnp.minimum(b0 + 3, last)
        pltpu.async_copy(x_hbm.at[pl.ds(nxt, 1)], slab1, sem1)

        pltpu.sync_copy(pair_v, masked_hbm.at[pl.ds((b0 - BT) * D, 2 * D)])

        idx_vec = jnp.where(iota16 == 2 * t, cand0, idx_vec)
        return jnp.where(iota16 == 2 * t + 1, cand1, idx_vec)

    idx_vec = lax.fori_loop(0, RW // 2, pair_body, jnp.zeros((L,), jnp.int32))
    _wait(slab0, sem0)
    _wait(slab1, sem1)

    idx_stage[pl.ds(0, L)] = idx_vec
    pltpu.sync_copy(idx_stage, idx_hbm.at[pl.ds(wid * L, L)])


@jax.jit
def kernel(x):
    xt = jnp.transpose(x, (0, 2, 1))
    sc_masked_flat, sc_idx_pad = _sc_kernel(xt)
    tc_masked, tc_idx = _tc_kernel(xt)
    masked = jnp.concatenate(
        [tc_masked, sc_masked_flat.reshape(NSC, D)], axis=0)
    sc_idx = sc_idx_pad.reshape(NW, L)[:, :RW].reshape(NSC)
    idx = jnp.concatenate([tc_idx.reshape(BT), sc_idx], axis=0)
    return masked[:, None, :], idx
